# baseline (speedup 1.0000x reference)
import jax
import jax.numpy as jnp
from jax.experimental import pallas as pl
from jax.experimental.pallas import tpu as pltpu

_BM = 400


def _spmm_body(x_ref, a_ref, o_ref):
    o_ref[...] = jnp.dot(a_ref[...], x_ref[...],
                         preferred_element_type=jnp.float32)


def kernel(input, adj):
    M, K = adj.shape
    N = input.shape[1]
    support = pl.pallas_call(
        _spmm_body,
        grid=(M // _BM,),
        in_specs=[
            pl.BlockSpec((K, N), lambda i: (0, 0)),
            pl.BlockSpec((_BM, K), lambda i: (i, 0)),
        ],
        out_specs=pl.BlockSpec((_BM, N), lambda i: (i, 0)),
        out_shape=jax.ShapeDtypeStruct((M, N), jnp.float32),
        compiler_params=pltpu.CompilerParams(
            dimension_semantics=("arbitrary",),
        ),
    )(input, adj)
    return (support, support)


# speedup vs baseline: 1.0063x; 1.0063x over previous
import jax
import jax.numpy as jnp
from jax.experimental import pallas as pl
from jax.experimental.pallas import tpu as pltpu

_BM = 200


def _spmm_body(x_ref, a_ref, o_ref):
    o_ref[...] = jnp.dot(a_ref[...], x_ref[...],
                         preferred_element_type=jnp.float32)


def kernel(input, adj):
    M, K = adj.shape
    N = input.shape[1]
    support = pl.pallas_call(
        _spmm_body,
        grid=(M // _BM,),
        in_specs=[
            pl.BlockSpec((K, N), lambda i: (0, 0)),
            pl.BlockSpec((_BM, K), lambda i: (i, 0)),
        ],
        out_specs=pl.BlockSpec((_BM, N), lambda i: (i, 0)),
        out_shape=jax.ShapeDtypeStruct((M, N), jnp.float32),
        compiler_params=pltpu.CompilerParams(
            dimension_semantics=("arbitrary",),
        ),
    )(input, adj)
    return (support, support)


# speedup vs baseline: 1.0448x; 1.0383x over previous
import jax
import jax.numpy as jnp
from jax.experimental import pallas as pl
from jax.experimental.pallas import tpu as pltpu

_BM = 200


def _spmm_body(x_ref, a_ref, o1_ref, o2_ref):
    s = jnp.dot(a_ref[...], x_ref[...], preferred_element_type=jnp.float32)
    o1_ref[...] = s
    o2_ref[...] = s


def kernel(input, adj):
    M, K = adj.shape
    N = input.shape[1]
    out_spec = pl.BlockSpec((_BM, N), lambda i: (i, 0))
    out_shape = jax.ShapeDtypeStruct((M, N), jnp.float32)
    s1, s2 = pl.pallas_call(
        _spmm_body,
        grid=(M // _BM,),
        in_specs=[
            pl.BlockSpec((K, N), lambda i: (0, 0)),
            pl.BlockSpec((_BM, K), lambda i: (i, 0)),
        ],
        out_specs=[out_spec, out_spec],
        out_shape=[out_shape, out_shape],
        compiler_params=pltpu.CompilerParams(
            dimension_semantics=("arbitrary",),
        ),
    )(input, adj)
    return (s1, s2)
